# baseline (speedup 1.0000x reference)
import functools

import jax
import jax.numpy as jnp
from jax import lax
from jax.experimental import pallas as pl
from jax.experimental.pallas import tpu as pltpu
from jax.experimental.pallas import tpu_sc as plsc


TOPK = 2
TBLK = 128
HBLK = 512
NC, NS = 2, 16
NW = NC * NS
DISP_CH = 32
COMB_CH = 32


RBLK = 512


def _route_body(cap, n_exp, nb, x_ref, wg_ref, e0_ref, e1_ref, l0_ref, l1_ref,
                g0_ref, g1_ref, offs_ref, fil_ref, cnts_ref):
    b = pl.program_id(0)
    xb = x_ref[...]
    logits = jnp.dot(xb, wg_ref[...], preferred_element_type=jnp.float32)
    iota_e = lax.broadcasted_iota(jnp.int32, logits.shape, 1)
    m1 = jnp.max(logits, axis=1, keepdims=True)
    i1 = jnp.min(jnp.where(logits == m1, iota_e, n_exp), axis=1, keepdims=True)
    l2 = jnp.where(iota_e == i1, -jnp.inf, logits)
    m2 = jnp.max(l2, axis=1, keepdims=True)
    i2 = jnp.min(jnp.where(l2 == m2, iota_e, n_exp), axis=1, keepdims=True)
    q = jnp.exp(m2 - m1)
    g0_ref[...] = (1.0 / (1.0 + q)).reshape(1, 1, RBLK)
    g1_ref[...] = (q / (1.0 + q)).reshape(1, 1, RBLK)
    e0_ref[...] = i1.reshape(1, 1, RBLK)
    e1_ref[...] = i2.reshape(1, 1, RBLK)

    ri = lax.broadcasted_iota(jnp.int32, (RBLK, RBLK), 0)
    ci = lax.broadcasted_iota(jnp.int32, (RBLK, RBLK), 1)
    tril = (ci < ri).astype(jnp.float32)
    mask0 = (iota_e == i1).astype(jnp.float32)
    mask1 = (iota_e == i2).astype(jnp.float32)
    lin0 = jnp.dot(tril, mask0, preferred_element_type=jnp.float32)
    lin1 = jnp.dot(tril, mask1, preferred_element_type=jnp.float32)
    l0_ref[...] = jnp.sum(lin0 * mask0, axis=1).astype(jnp.int32).reshape(
        1, 1, RBLK)
    l1_ref[...] = jnp.sum(lin1 * mask1, axis=1).astype(jnp.int32).reshape(
        1, 1, RBLK)
    cnts_ref[pl.ds(b, 1), :] = jnp.sum(mask0, axis=0, keepdims=True)
    cnts_ref[pl.ds(nb + b, 1), :] = jnp.sum(mask1, axis=0, keepdims=True)

    @pl.when(b == nb - 1)
    def _():
        cn = cnts_ref[...]
        r2 = lax.broadcasted_iota(jnp.int32, (2 * nb, 2 * nb), 0)
        c2 = lax.broadcasted_iota(jnp.int32, (2 * nb, 2 * nb), 1)
        trils = (c2 < r2).astype(jnp.float32)
        offs = jnp.dot(trils, cn, preferred_element_type=jnp.float32)
        pad = jnp.zeros((2 * nb, 16 - n_exp), jnp.float32)
        offs_ref[...] = jnp.concatenate([offs, pad], axis=1).reshape(
            2 * nb, 1, 16)
        total = jnp.sum(cn, axis=0, keepdims=True)
        fil_ref[...] = jnp.minimum(total, float(cap)).reshape(1, 1, n_exp)


def _route(x2d, wg, cap):
    t, d = x2d.shape
    n_exp = wg.shape[1]
    nb = t // RBLK
    tok3 = jax.ShapeDtypeStruct((nb, 1, RBLK), jnp.int32)
    tok3f = jax.ShapeDtypeStruct((nb, 1, RBLK), jnp.float32)
    out_shape = [
        tok3, tok3,
        tok3, tok3,
        tok3f, tok3f,
        jax.ShapeDtypeStruct((TOPK * nb, 1, 16), jnp.float32),
        jax.ShapeDtypeStruct((1, 1, n_exp), jnp.float32),
    ]
    tok_spec = pl.BlockSpec((1, 1, RBLK), lambda b: (b, 0, 0))
    return pl.pallas_call(
        functools.partial(_route_body, cap, n_exp, nb),
        grid=(nb,),
        in_specs=[
            pl.BlockSpec((RBLK, d), lambda b: (b, 0)),
            pl.BlockSpec((d, n_exp), lambda b: (0, 0)),
        ],
        out_specs=[
            tok_spec, tok_spec, tok_spec, tok_spec, tok_spec, tok_spec,
            pl.BlockSpec((TOPK * nb, 1, 16), lambda b: (0, 0, 0)),
            pl.BlockSpec((1, 1, n_exp), lambda b: (0, 0, 0)),
        ],
        out_shape=out_shape,
        scratch_shapes=[pltpu.VMEM((TOPK * nb, n_exp), jnp.float32)],
    )(x2d, wg)


def _dispatch(x2d, e0, e1, l0, l1, g0, g1, offs, cap, n_exp, nb_r):
    t, d = x2d.shape
    rows_out = (n_exp + 1) * cap
    tok_per_w = t // NW
    nch = tok_per_w // DISP_CH
    blk_per_r = RBLK // tok_per_w
    mesh = plsc.VectorSubcoreMesh(core_axis_name="c", subcore_axis_name="s")

    @functools.partial(
        pl.kernel,
        out_type=[
            jax.ShapeDtypeStruct((rows_out, d), jnp.int32),
            jax.ShapeDtypeStruct((t // 16, 16), jnp.int32),
            jax.ShapeDtypeStruct((t // 16, 16), jnp.int32),
            jax.ShapeDtypeStruct((t // 16, 16), jnp.float32),
            jax.ShapeDtypeStruct((t // 16, 16), jnp.float32),
        ],
        mesh=mesh,
        scratch_types=[
            pltpu.VMEM((16,), jnp.float32),
            pltpu.VMEM((16,), jnp.float32),
            pltpu.VMEM((8, 16), jnp.int32),
            pltpu.VMEM((8, 16), jnp.int32),
            pltpu.VMEM((8, 16), jnp.int32),
            pltpu.VMEM((8, 16), jnp.int32),
            pltpu.VMEM((8, 16), jnp.float32),
            pltpu.VMEM((8, 16), jnp.float32),
            pltpu.VMEM((2, DISP_CH, 512), jnp.int32),
            pltpu.VMEM((16, 16), jnp.int32),
            pltpu.VMEM((16, 16), jnp.float32),
            pltpu.VMEM((8, 16), jnp.int32),
            pltpu.SemaphoreType.DMA,
            pltpu.SemaphoreType.DMA,
            pltpu.SemaphoreType.DMA,
            pltpu.SemaphoreType.DMA,
            pltpu.SemaphoreType.DMA,
            pltpu.SemaphoreType.DMA,
        ],
    )
    def disp_k(x_hbm, e0_hbm, e1_hbm, l0_hbm, l1_hbm, g0_hbm, g1_hbm,
               offs_hbm, out_hbm, pc0_hbm, pc1_hbm, ge0_hbm, ge1_hbm,
               off0_v, off1_v, e0_v, e1_v, l0_v, l1_v, g0_v, g1_v,
               rows_v, pc_acc, ge_acc, pd_buf, sxa, sxb, s0a, s1a, s0b, s1b):
        wid = lax.axis_index("s") * NC + lax.axis_index("c")
        rb = wid // blk_per_r
        rowb = wid * (tok_per_w // 16)
        pltpu.sync_copy(offs_hbm.at[pl.ds(rb * 16, 16)], off0_v)
        pltpu.sync_copy(offs_hbm.at[pl.ds((nb_r + rb) * 16, 16)], off1_v)
        pltpu.sync_copy(e0_hbm.at[pl.ds(rowb, 8)], e0_v)
        pltpu.sync_copy(e1_hbm.at[pl.ds(rowb, 8)], e1_v)
        pltpu.sync_copy(l0_hbm.at[pl.ds(rowb, 8)], l0_v)
        pltpu.sync_copy(l1_hbm.at[pl.ds(rowb, 8)], l1_v)
        pltpu.sync_copy(g0_hbm.at[pl.ds(rowb, 8)], g0_v)
        pltpu.sync_copy(g1_hbm.at[pl.ds(rowb, 8)], g1_v)
        offs_regs = (off0_v[...], off1_v[...])
        elg = ((e0_v, l0_v, g0_v), (e1_v, l1_v, g1_v))
        xsems = (sxa, sxb)
        ssems = ((s0a, s1a), (s0b, s1b))

        def startx(ch):
            bi = ch % 2
            return pltpu.async_copy(
                x_hbm.at[pl.ds(wid * tok_per_w + ch * DISP_CH, DISP_CH)],
                rows_v.at[bi], xsems[bi])

        xp = startx(0)
        scat_prev = []
        for ch in range(nch):
            bi = ch % 2
            xp.wait()
            if ch + 1 < nch:
                for c in scat_prev:
                    c.wait()
                scat_prev = []
                xp = startx(ch + 1)
            scat_cur = []
            for j in range(TOPK):
                ev_r, lv_r, gv_r = elg[j]
                for grp in range(DISP_CH // 16):
                    row = ch * (DISP_CH // 16) + grp
                    ev = ev_r[row]
                    lv = lv_r[row]
                    gv = gv_r[row]
                    offv = offs_regs[j].at[ev].get(mode="promise_in_bounds")
                    loc = lv + offv.astype(jnp.int32)
                    valid = loc < cap
                    pos = ev * cap + jnp.minimum(loc, cap - 1)
                    pdv = jnp.where(valid, pos, n_exp * cap)
                    pcv = jnp.where(valid, pos, 0)
                    gev = jnp.where(valid, gv, 0.0)
                    pc_acc[j * 8 + row] = pcv
                    ge_acc[j * 8 + row] = gev
                    kk = bi * 4 + j * 2 + grp
                    pd_buf[kk] = pdv
                    scat_cur.append(pltpu.async_copy(
                        rows_v.at[bi, pl.ds(grp * 16, 16)],
                        out_hbm.at[pd_buf.at[kk]], ssems[bi][j]))
            scat_prev = scat_prev + scat_cur
        for c in scat_prev:
            c.wait()
        pltpu.sync_copy(pc_acc.at[pl.ds(0, 8)],
                        pc0_hbm.at[pl.ds(wid * 8, 8)])
        pltpu.sync_copy(pc_acc.at[pl.ds(8, 8)],
                        pc1_hbm.at[pl.ds(wid * 8, 8)])
        pltpu.sync_copy(ge_acc.at[pl.ds(0, 8)],
                        ge0_hbm.at[pl.ds(wid * 8, 8)])
        pltpu.sync_copy(ge_acc.at[pl.ds(8, 8)],
                        ge1_hbm.at[pl.ds(wid * 8, 8)])

    return disp_k(x2d, e0, e1, l0, l1, g0, g1, offs)


def _ffn_body(nh, d_ref, w1_ref, b1_ref, w2_ref, b2_ref, fil_ref, y_ref):
    h = pl.program_id(1)
    w1b = w1_ref[0].astype(jnp.bfloat16)
    hb = jnp.dot(d_ref[...], w1b, preferred_element_type=jnp.float32)
    hb = jnp.maximum(hb + b1_ref[0], 0.0)
    w2b = w2_ref[0].astype(jnp.bfloat16)
    yb = jnp.dot(hb.astype(jnp.bfloat16), w2b,
                 preferred_element_type=jnp.float32)

    @pl.when(h == 0)
    def _():
        y_ref[...] = yb + b2_ref[0]

    @pl.when(jnp.logical_and(h != 0, h != nh - 1))
    def _():
        y_ref[...] += yb

    @pl.when(h == nh - 1)
    def _():
        acc = y_ref[...] + yb
        rows = lax.broadcasted_iota(jnp.int32, acc.shape, 0)
        y_ref[...] = jnp.where(rows < fil_ref[0, 0, 0], acc, 0.0)


def _ffn(disp, w1, b1, w2, b2, filled, cap):
    n_exp, d, hdim = w1.shape
    nh = hdim // HBLK
    return pl.pallas_call(
        functools.partial(_ffn_body, nh),
        grid=(n_exp, nh),
        in_specs=[
            pl.BlockSpec((cap, d), lambda e, h: (e, 0)),
            pl.BlockSpec((1, d, HBLK), lambda e, h: (e, 0, h)),
            pl.BlockSpec((1, 1, HBLK), lambda e, h: (e, 0, h)),
            pl.BlockSpec((1, HBLK, d), lambda e, h: (e, h, 0)),
            pl.BlockSpec((1, 1, d), lambda e, h: (e, 0, 0)),
            pl.BlockSpec((1, 1, 1), lambda e, h: (e, 0, 0)),
        ],
        out_specs=pl.BlockSpec((cap, d), lambda e, h: (e, 0)),
        out_shape=jax.ShapeDtypeStruct((n_exp * cap, d), jnp.float32),
    )(disp, w1, b1.reshape(n_exp, 1, hdim), w2, b2.reshape(n_exp, 1, d),
      filled)


def _combine_sc(yexp, pc0, pc1, ge0, ge1, t):
    _, d = yexp.shape
    tok_per_w = t // NW
    nch = tok_per_w // 16
    mesh = plsc.VectorSubcoreMesh(core_axis_name="c", subcore_axis_name="s")

    @functools.partial(
        pl.kernel,
        out_type=jax.ShapeDtypeStruct((t, d), jnp.float32),
        mesh=mesh,
        scratch_types=[
            pltpu.VMEM((nch, 16), jnp.int32),
            pltpu.VMEM((nch, 16), jnp.int32),
            pltpu.VMEM((nch, 16), jnp.float32),
            pltpu.VMEM((nch, 16), jnp.float32),
            pltpu.VMEM((2, 16, 1024), jnp.float32),
            pltpu.VMEM((2, 16, 1024), jnp.float32),
            pltpu.VMEM((2, 16, 1024), jnp.float32),
            pltpu.SemaphoreType.DMA,
            pltpu.SemaphoreType.DMA,
            pltpu.SemaphoreType.DMA,
            pltpu.SemaphoreType.DMA,
            pltpu.SemaphoreType.DMA,
        ],
    )
    def comb_k(y_hbm, p0_hbm, p1_hbm, g0_hbm, g1_hbm, out_hbm,
               i0_v, i1_v, g0_v, g1_v, r0_v, r1_v, ro_v, sa0, sa1, sb0, sb1,
               so):
        wid = lax.axis_index("s") * NC + lax.axis_index("c")
        rowb = wid * nch
        pltpu.sync_copy(p0_hbm.at[pl.ds(rowb, nch)], i0_v)
        pltpu.sync_copy(p1_hbm.at[pl.ds(rowb, nch)], i1_v)
        pltpu.sync_copy(g0_hbm.at[pl.ds(rowb, nch)], g0_v)
        pltpu.sync_copy(g1_hbm.at[pl.ds(rowb, nch)], g1_v)
        sems = ((sa0, sa1), (sb0, sb1))

        def start(ch):
            bi = ch % 2
            c0 = pltpu.async_copy(y_hbm.at[i0_v.at[ch]], r0_v.at[bi],
                                  sems[bi][0])
            c1 = pltpu.async_copy(y_hbm.at[i1_v.at[ch]], r1_v.at[bi],
                                  sems[bi][1])
            return (c0, c1)

        pending = start(0)
        out_pending = None
        for ch in range(nch):
            bi = ch % 2
            pending[0].wait()
            pending[1].wait()
            if ch + 1 < nch:
                pending = start(ch + 1)
            ga16 = g0_v[ch]
            gb16 = g1_v[ch]
            if out_pending is not None:
                out_pending.wait()
            for lane in range(16):
                lidx = jnp.full((16,), lane, jnp.int32)
                va = ga16.at[lidx].get(mode="promise_in_bounds")
                vb = gb16.at[lidx].get(mode="promise_in_bounds")

                def col_body(k, c2, lane=lane, va=va, vb=vb, bi=bi):
                    for u in range(4):
                        sl = pl.ds(k * 64 + u * 16, 16)
                        ro_v[bi, lane, sl] = (va * r0_v[bi, lane, sl]
                                              + vb * r1_v[bi, lane, sl])
                    return c2

                lax.fori_loop(0, d // 64, col_body, 0)
            out_pending = pltpu.async_copy(
                ro_v.at[bi], out_hbm.at[pl.ds(wid * tok_per_w + ch * 16, 16)],
                so)
        out_pending.wait()

    return comb_k(yexp, pc0, pc1, ge0, ge1)


def kernel(x, wg, w1, b1, w2, b2):
    orig_shape = x.shape
    d = orig_shape[-1]
    x2d = x.reshape(-1, d)
    t = x2d.shape[0]
    n_exp = w1.shape[0]
    cap = -(-TOPK * t // n_exp)

    e0, e1, l0, l1, g0, g1, offs, filled = _route(x2d, wg, cap)
    nb_r = t // RBLK
    t16 = t // 16
    x_i32 = lax.bitcast_convert_type(
        x2d.astype(jnp.bfloat16).reshape(t, d // 2, 2), jnp.int32)
    disp, pc0, pc1, ge0, ge1 = _dispatch(
        x_i32, e0.reshape(t16, 16), e1.reshape(t16, 16), l0.reshape(t16, 16),
        l1.reshape(t16, 16), g0.reshape(t16, 16), g1.reshape(t16, 16),
        offs.reshape(-1), cap, n_exp, nb_r)
    disp_bf = lax.bitcast_convert_type(disp, jnp.bfloat16).reshape(-1, d)
    yexp = _ffn(disp_bf, w1, b1, w2, b2, filled.reshape(n_exp, 1, 1), cap)
    out = _combine_sc(yexp, pc0, pc1, ge0, ge1, t)
    return out.reshape(orig_shape)


# speedup vs baseline: 2.1104x; 2.1104x over previous
import functools

import jax
import jax.numpy as jnp
from jax import lax
from jax.experimental import pallas as pl
from jax.experimental.pallas import tpu as pltpu
from jax.experimental.pallas import tpu_sc as plsc


TOPK = 2
TBLK = 128
HBLK = 512
NC, NS = 2, 16
NW = NC * NS
DISP_CH = 32
COMB_CH = 32


RBLK = 512


def _route_body(cap, n_exp, nb, x_ref, wg_ref, e0_ref, e1_ref, l0_ref, l1_ref,
                g0_ref, g1_ref, offs_ref, fil_ref, cnts_ref):
    b = pl.program_id(0)
    xb = x_ref[...]
    logits = jnp.dot(xb, wg_ref[...], preferred_element_type=jnp.float32)
    iota_e = lax.broadcasted_iota(jnp.int32, logits.shape, 1)
    m1 = jnp.max(logits, axis=1, keepdims=True)
    i1 = jnp.min(jnp.where(logits == m1, iota_e, n_exp), axis=1, keepdims=True)
    l2 = jnp.where(iota_e == i1, -jnp.inf, logits)
    m2 = jnp.max(l2, axis=1, keepdims=True)
    i2 = jnp.min(jnp.where(l2 == m2, iota_e, n_exp), axis=1, keepdims=True)
    q = jnp.exp(m2 - m1)
    g0_ref[...] = (1.0 / (1.0 + q)).reshape(1, 1, RBLK)
    g1_ref[...] = (q / (1.0 + q)).reshape(1, 1, RBLK)
    e0_ref[...] = i1.reshape(1, 1, RBLK)
    e1_ref[...] = i2.reshape(1, 1, RBLK)

    ri = lax.broadcasted_iota(jnp.int32, (RBLK, RBLK), 0)
    ci = lax.broadcasted_iota(jnp.int32, (RBLK, RBLK), 1)
    tril = (ci < ri).astype(jnp.float32)
    mask0 = (iota_e == i1).astype(jnp.float32)
    mask1 = (iota_e == i2).astype(jnp.float32)
    lin0 = jnp.dot(tril, mask0, preferred_element_type=jnp.float32)
    lin1 = jnp.dot(tril, mask1, preferred_element_type=jnp.float32)
    l0_ref[...] = jnp.sum(lin0 * mask0, axis=1).astype(jnp.int32).reshape(
        1, 1, RBLK)
    l1_ref[...] = jnp.sum(lin1 * mask1, axis=1).astype(jnp.int32).reshape(
        1, 1, RBLK)
    cnts_ref[pl.ds(b, 1), :] = jnp.sum(mask0, axis=0, keepdims=True)
    cnts_ref[pl.ds(nb + b, 1), :] = jnp.sum(mask1, axis=0, keepdims=True)

    @pl.when(b == nb - 1)
    def _():
        cn = cnts_ref[...]
        r2 = lax.broadcasted_iota(jnp.int32, (2 * nb, 2 * nb), 0)
        c2 = lax.broadcasted_iota(jnp.int32, (2 * nb, 2 * nb), 1)
        trils = (c2 < r2).astype(jnp.float32)
        offs = jnp.dot(trils, cn, preferred_element_type=jnp.float32)
        pad = jnp.zeros((2 * nb, 16 - n_exp), jnp.float32)
        offs_ref[...] = jnp.concatenate([offs, pad], axis=1).reshape(
            2 * nb, 1, 16)
        total = jnp.sum(cn, axis=0, keepdims=True)
        fil_ref[...] = jnp.minimum(total, float(cap)).reshape(1, 1, n_exp)


def _route(x2d, wg, cap):
    t, d = x2d.shape
    n_exp = wg.shape[1]
    nb = t // RBLK
    tok3 = jax.ShapeDtypeStruct((nb, 1, RBLK), jnp.int32)
    tok3f = jax.ShapeDtypeStruct((nb, 1, RBLK), jnp.float32)
    out_shape = [
        tok3, tok3,
        tok3, tok3,
        tok3f, tok3f,
        jax.ShapeDtypeStruct((TOPK * nb, 1, 16), jnp.float32),
        jax.ShapeDtypeStruct((1, 1, n_exp), jnp.float32),
    ]
    tok_spec = pl.BlockSpec((1, 1, RBLK), lambda b: (b, 0, 0))
    return pl.pallas_call(
        functools.partial(_route_body, cap, n_exp, nb),
        grid=(nb,),
        in_specs=[
            pl.BlockSpec((RBLK, d), lambda b: (b, 0)),
            pl.BlockSpec((d, n_exp), lambda b: (0, 0)),
        ],
        out_specs=[
            tok_spec, tok_spec, tok_spec, tok_spec, tok_spec, tok_spec,
            pl.BlockSpec((TOPK * nb, 1, 16), lambda b: (0, 0, 0)),
            pl.BlockSpec((1, 1, n_exp), lambda b: (0, 0, 0)),
        ],
        out_shape=out_shape,
        scratch_shapes=[pltpu.VMEM((TOPK * nb, n_exp), jnp.float32)],
    )(x2d, wg)


def _dispatch(x2d, e0, e1, l0, l1, g0, g1, offs, cap, n_exp, nb_r):
    t, d = x2d.shape
    rows_out = (n_exp + 1) * cap
    tok_per_w = t // NW
    nch = tok_per_w // DISP_CH
    blk_per_r = RBLK // tok_per_w
    mesh = plsc.VectorSubcoreMesh(core_axis_name="c", subcore_axis_name="s")

    @functools.partial(
        pl.kernel,
        out_type=[
            jax.ShapeDtypeStruct((rows_out, d), jnp.float32),
            jax.ShapeDtypeStruct((t // 16, 16), jnp.int32),
            jax.ShapeDtypeStruct((t // 16, 16), jnp.int32),
            jax.ShapeDtypeStruct((t // 16, 16), jnp.float32),
            jax.ShapeDtypeStruct((t // 16, 16), jnp.float32),
        ],
        mesh=mesh,
        scratch_types=[
            pltpu.VMEM((16,), jnp.float32),
            pltpu.VMEM((16,), jnp.float32),
            pltpu.VMEM((8, 16), jnp.int32),
            pltpu.VMEM((8, 16), jnp.int32),
            pltpu.VMEM((8, 16), jnp.int32),
            pltpu.VMEM((8, 16), jnp.int32),
            pltpu.VMEM((8, 16), jnp.float32),
            pltpu.VMEM((8, 16), jnp.float32),
            pltpu.VMEM((2, DISP_CH, 1024), jnp.float32),
            pltpu.VMEM((16, 16), jnp.int32),
            pltpu.VMEM((16, 16), jnp.float32),
            pltpu.VMEM((8, 16), jnp.int32),
            pltpu.SemaphoreType.DMA,
            pltpu.SemaphoreType.DMA,
            pltpu.SemaphoreType.DMA,
            pltpu.SemaphoreType.DMA,
            pltpu.SemaphoreType.DMA,
            pltpu.SemaphoreType.DMA,
        ],
    )
    def disp_k(x_hbm, e0_hbm, e1_hbm, l0_hbm, l1_hbm, g0_hbm, g1_hbm,
               offs_hbm, out_hbm, pc0_hbm, pc1_hbm, ge0_hbm, ge1_hbm,
               off0_v, off1_v, e0_v, e1_v, l0_v, l1_v, g0_v, g1_v,
               rows_v, pc_acc, ge_acc, pd_buf, sxa, sxb, s0a, s1a, s0b, s1b):
        wid = lax.axis_index("s") * NC + lax.axis_index("c")
        rb = wid // blk_per_r
        rowb = wid * (tok_per_w // 16)
        pltpu.sync_copy(offs_hbm.at[pl.ds(rb * 16, 16)], off0_v)
        pltpu.sync_copy(offs_hbm.at[pl.ds((nb_r + rb) * 16, 16)], off1_v)
        pltpu.sync_copy(e0_hbm.at[pl.ds(rowb, 8)], e0_v)
        pltpu.sync_copy(e1_hbm.at[pl.ds(rowb, 8)], e1_v)
        pltpu.sync_copy(l0_hbm.at[pl.ds(rowb, 8)], l0_v)
        pltpu.sync_copy(l1_hbm.at[pl.ds(rowb, 8)], l1_v)
        pltpu.sync_copy(g0_hbm.at[pl.ds(rowb, 8)], g0_v)
        pltpu.sync_copy(g1_hbm.at[pl.ds(rowb, 8)], g1_v)
        offs_regs = (off0_v[...], off1_v[...])
        elg = ((e0_v, l0_v, g0_v), (e1_v, l1_v, g1_v))
        xsems = (sxa, sxb)
        ssems = ((s0a, s1a), (s0b, s1b))

        def startx(ch):
            bi = ch % 2
            return pltpu.async_copy(
                x_hbm.at[pl.ds(wid * tok_per_w + ch * DISP_CH, DISP_CH)],
                rows_v.at[bi], xsems[bi])

        xp = startx(0)
        scat_prev = []
        for ch in range(nch):
            bi = ch % 2
            xp.wait()
            if ch + 1 < nch:
                for c in scat_prev:
                    c.wait()
                scat_prev = []
                xp = startx(ch + 1)
            scat_cur = []
            for j in range(TOPK):
                ev_r, lv_r, gv_r = elg[j]
                for grp in range(DISP_CH // 16):
                    row = ch * (DISP_CH // 16) + grp
                    ev = ev_r[row]
                    lv = lv_r[row]
                    gv = gv_r[row]
                    offv = offs_regs[j].at[ev].get(mode="promise_in_bounds")
                    loc = lv + offv.astype(jnp.int32)
                    valid = loc < cap
                    pos = ev * cap + jnp.minimum(loc, cap - 1)
                    pdv = jnp.where(valid, pos, n_exp * cap)
                    pcv = jnp.where(valid, pos, 0)
                    gev = jnp.where(valid, gv, 0.0)
                    pc_acc[j * 8 + row] = pcv
                    ge_acc[j * 8 + row] = gev
                    kk = bi * 4 + j * 2 + grp
                    pd_buf[kk] = pdv
                    scat_cur.append(pltpu.async_copy(
                        rows_v.at[bi, pl.ds(grp * 16, 16)],
                        out_hbm.at[pd_buf.at[kk]], ssems[bi][j]))
            scat_prev = scat_prev + scat_cur
        for c in scat_prev:
            c.wait()
        pltpu.sync_copy(pc_acc.at[pl.ds(0, 8)],
                        pc0_hbm.at[pl.ds(wid * 8, 8)])
        pltpu.sync_copy(pc_acc.at[pl.ds(8, 8)],
                        pc1_hbm.at[pl.ds(wid * 8, 8)])
        pltpu.sync_copy(ge_acc.at[pl.ds(0, 8)],
                        ge0_hbm.at[pl.ds(wid * 8, 8)])
        pltpu.sync_copy(ge_acc.at[pl.ds(8, 8)],
                        ge1_hbm.at[pl.ds(wid * 8, 8)])

    return disp_k(x2d, e0, e1, l0, l1, g0, g1, offs)


def _ffn_body(nh, d_ref, w1_ref, b1_ref, w2_ref, b2_ref, fil_ref, y_ref):
    h = pl.program_id(1)
    w1b = w1_ref[0].astype(jnp.bfloat16)
    hb = jnp.dot(d_ref[...].astype(jnp.bfloat16), w1b,
                 preferred_element_type=jnp.float32)
    hb = jnp.maximum(hb + b1_ref[0], 0.0)
    w2b = w2_ref[0].astype(jnp.bfloat16)
    yb = jnp.dot(hb.astype(jnp.bfloat16), w2b,
                 preferred_element_type=jnp.float32)

    @pl.when(h == 0)
    def _():
        y_ref[...] = yb + b2_ref[0]

    @pl.when(jnp.logical_and(h != 0, h != nh - 1))
    def _():
        y_ref[...] += yb

    @pl.when(h == nh - 1)
    def _():
        acc = y_ref[...] + yb
        rows = lax.broadcasted_iota(jnp.int32, acc.shape, 0)
        y_ref[...] = jnp.where(rows < fil_ref[0, 0, 0], acc, 0.0)


def _ffn(disp, w1, b1, w2, b2, filled, cap):
    n_exp, d, hdim = w1.shape
    nh = hdim // HBLK
    return pl.pallas_call(
        functools.partial(_ffn_body, nh),
        grid=(n_exp, nh),
        in_specs=[
            pl.BlockSpec((cap, d), lambda e, h: (e, 0)),
            pl.BlockSpec((1, d, HBLK), lambda e, h: (e, 0, h)),
            pl.BlockSpec((1, 1, HBLK), lambda e, h: (e, 0, h)),
            pl.BlockSpec((1, HBLK, d), lambda e, h: (e, h, 0)),
            pl.BlockSpec((1, 1, d), lambda e, h: (e, 0, 0)),
            pl.BlockSpec((1, 1, 1), lambda e, h: (e, 0, 0)),
        ],
        out_specs=pl.BlockSpec((cap, d), lambda e, h: (e, 0)),
        out_shape=jax.ShapeDtypeStruct((n_exp * cap, d), jnp.float32),
    )(disp, w1, b1.reshape(n_exp, 1, hdim), w2, b2.reshape(n_exp, 1, d),
      filled)


def _combine_sc(yexp, pc0, pc1, ge0, ge1, t):
    _, d = yexp.shape
    tok_per_w = t // NW
    nch = tok_per_w // 16
    mesh = plsc.VectorSubcoreMesh(core_axis_name="c", subcore_axis_name="s")

    @functools.partial(
        pl.kernel,
        out_type=jax.ShapeDtypeStruct((t, d), jnp.float32),
        mesh=mesh,
        scratch_types=[
            pltpu.VMEM((nch, 16), jnp.int32),
            pltpu.VMEM((nch, 16), jnp.int32),
            pltpu.VMEM((nch, 16), jnp.float32),
            pltpu.VMEM((nch, 16), jnp.float32),
            pltpu.VMEM((2, 16, 1024), jnp.float32),
            pltpu.VMEM((2, 16, 1024), jnp.float32),
            pltpu.VMEM((2, 16, 1024), jnp.float32),
            pltpu.SemaphoreType.DMA,
            pltpu.SemaphoreType.DMA,
            pltpu.SemaphoreType.DMA,
            pltpu.SemaphoreType.DMA,
            pltpu.SemaphoreType.DMA,
        ],
    )
    def comb_k(y_hbm, p0_hbm, p1_hbm, g0_hbm, g1_hbm, out_hbm,
               i0_v, i1_v, g0_v, g1_v, r0_v, r1_v, ro_v, sa0, sa1, sb0, sb1,
               so):
        wid = lax.axis_index("s") * NC + lax.axis_index("c")
        rowb = wid * nch
        pltpu.sync_copy(p0_hbm.at[pl.ds(rowb, nch)], i0_v)
        pltpu.sync_copy(p1_hbm.at[pl.ds(rowb, nch)], i1_v)
        pltpu.sync_copy(g0_hbm.at[pl.ds(rowb, nch)], g0_v)
        pltpu.sync_copy(g1_hbm.at[pl.ds(rowb, nch)], g1_v)
        sems = ((sa0, sa1), (sb0, sb1))

        def start(ch):
            bi = ch % 2
            c0 = pltpu.async_copy(y_hbm.at[i0_v.at[ch]], r0_v.at[bi],
                                  sems[bi][0])
            c1 = pltpu.async_copy(y_hbm.at[i1_v.at[ch]], r1_v.at[bi],
                                  sems[bi][1])
            return (c0, c1)

        pending = start(0)
        out_pending = None
        for ch in range(nch):
            bi = ch % 2
            pending[0].wait()
            pending[1].wait()
            if ch + 1 < nch:
                pending = start(ch + 1)
            ga16 = g0_v[ch]
            gb16 = g1_v[ch]
            if out_pending is not None:
                out_pending.wait()
            for lane in range(16):
                lidx = jnp.full((16,), lane, jnp.int32)
                va = ga16.at[lidx].get(mode="promise_in_bounds")
                vb = gb16.at[lidx].get(mode="promise_in_bounds")

                def col_body(k, c2, lane=lane, va=va, vb=vb, bi=bi):
                    for u in range(4):
                        sl = pl.ds(k * 64 + u * 16, 16)
                        ro_v[bi, lane, sl] = (va * r0_v[bi, lane, sl]
                                              + vb * r1_v[bi, lane, sl])
                    return c2

                lax.fori_loop(0, d // 64, col_body, 0)
            out_pending = pltpu.async_copy(
                ro_v.at[bi], out_hbm.at[pl.ds(wid * tok_per_w + ch * 16, 16)],
                so)
        out_pending.wait()

    return comb_k(yexp, pc0, pc1, ge0, ge1)


def kernel(x, wg, w1, b1, w2, b2):
    orig_shape = x.shape
    d = orig_shape[-1]
    x2d = x.reshape(-1, d)
    t = x2d.shape[0]
    n_exp = w1.shape[0]
    cap = -(-TOPK * t // n_exp)

    e0, e1, l0, l1, g0, g1, offs, filled = _route(x2d, wg, cap)
    nb_r = t // RBLK
    t16 = t // 16
    disp, pc0, pc1, ge0, ge1 = _dispatch(
        x2d, e0.reshape(t16, 16), e1.reshape(t16, 16), l0.reshape(t16, 16),
        l1.reshape(t16, 16), g0.reshape(t16, 16), g1.reshape(t16, 16),
        offs.reshape(-1), cap, n_exp, nb_r)
    yexp = _ffn(disp, w1, b1, w2, b2, filled.reshape(n_exp, 1, 1), cap)
    out = _combine_sc(yexp, pc0, pc1, ge0, ge1, t)
    return out.reshape(orig_shape)


# speedup vs baseline: 2.1830x; 1.0344x over previous
import functools

import jax
import jax.numpy as jnp
from jax import lax
from jax.experimental import pallas as pl
from jax.experimental.pallas import tpu as pltpu
from jax.experimental.pallas import tpu_sc as plsc


TOPK = 2
TBLK = 128
HBLK = 512
NC, NS = 2, 16
NW = NC * NS
DISP_CH = 32
COMB_CH = 32


RBLK = 512


def _route_body(cap, n_exp, nb, x_ref, wg_ref, e0_ref, e1_ref, l0_ref, l1_ref,
                g0_ref, g1_ref, offs_ref, fil_ref, cnts_ref):
    b = pl.program_id(0)
    xb = x_ref[...]
    logits = jnp.dot(xb, wg_ref[...], preferred_element_type=jnp.float32)
    iota_e = lax.broadcasted_iota(jnp.int32, logits.shape, 1)
    m1 = jnp.max(logits, axis=1, keepdims=True)
    i1 = jnp.min(jnp.where(logits == m1, iota_e, n_exp), axis=1, keepdims=True)
    l2 = jnp.where(iota_e == i1, -jnp.inf, logits)
    m2 = jnp.max(l2, axis=1, keepdims=True)
    i2 = jnp.min(jnp.where(l2 == m2, iota_e, n_exp), axis=1, keepdims=True)
    q = jnp.exp(m2 - m1)
    g0_ref[...] = (1.0 / (1.0 + q)).reshape(1, 1, RBLK)
    g1_ref[...] = (q / (1.0 + q)).reshape(1, 1, RBLK)
    e0_ref[...] = i1.reshape(1, 1, RBLK)
    e1_ref[...] = i2.reshape(1, 1, RBLK)

    ri = lax.broadcasted_iota(jnp.int32, (RBLK, RBLK), 0)
    ci = lax.broadcasted_iota(jnp.int32, (RBLK, RBLK), 1)
    tril = (ci < ri).astype(jnp.float32)
    mask0 = (iota_e == i1).astype(jnp.float32)
    mask1 = (iota_e == i2).astype(jnp.float32)
    lin0 = jnp.dot(tril, mask0, preferred_element_type=jnp.float32)
    lin1 = jnp.dot(tril, mask1, preferred_element_type=jnp.float32)
    l0_ref[...] = jnp.sum(lin0 * mask0, axis=1).astype(jnp.int32).reshape(
        1, 1, RBLK)
    l1_ref[...] = jnp.sum(lin1 * mask1, axis=1).astype(jnp.int32).reshape(
        1, 1, RBLK)
    cnts_ref[pl.ds(b, 1), :] = jnp.sum(mask0, axis=0, keepdims=True)
    cnts_ref[pl.ds(nb + b, 1), :] = jnp.sum(mask1, axis=0, keepdims=True)

    @pl.when(b == nb - 1)
    def _():
        cn = cnts_ref[...]
        r2 = lax.broadcasted_iota(jnp.int32, (2 * nb, 2 * nb), 0)
        c2 = lax.broadcasted_iota(jnp.int32, (2 * nb, 2 * nb), 1)
        trils = (c2 < r2).astype(jnp.float32)
        offs = jnp.dot(trils, cn, preferred_element_type=jnp.float32)
        pad = jnp.zeros((2 * nb, 16 - n_exp), jnp.float32)
        offs_ref[...] = jnp.concatenate([offs, pad], axis=1).reshape(
            2 * nb, 1, 16)
        total = jnp.sum(cn, axis=0, keepdims=True)
        fil_ref[...] = jnp.minimum(total, float(cap)).reshape(1, 1, n_exp)


def _route(x2d, wg, cap):
    t, d = x2d.shape
    n_exp = wg.shape[1]
    nb = t // RBLK
    tok3 = jax.ShapeDtypeStruct((nb, 1, RBLK), jnp.int32)
    tok3f = jax.ShapeDtypeStruct((nb, 1, RBLK), jnp.float32)
    out_shape = [
        tok3, tok3,
        tok3, tok3,
        tok3f, tok3f,
        jax.ShapeDtypeStruct((TOPK * nb, 1, 16), jnp.float32),
        jax.ShapeDtypeStruct((1, 1, n_exp), jnp.float32),
    ]
    tok_spec = pl.BlockSpec((1, 1, RBLK), lambda b: (b, 0, 0))
    return pl.pallas_call(
        functools.partial(_route_body, cap, n_exp, nb),
        grid=(nb,),
        in_specs=[
            pl.BlockSpec((RBLK, d), lambda b: (b, 0)),
            pl.BlockSpec((d, n_exp), lambda b: (0, 0)),
        ],
        out_specs=[
            tok_spec, tok_spec, tok_spec, tok_spec, tok_spec, tok_spec,
            pl.BlockSpec((TOPK * nb, 1, 16), lambda b: (0, 0, 0)),
            pl.BlockSpec((1, 1, n_exp), lambda b: (0, 0, 0)),
        ],
        out_shape=out_shape,
        scratch_shapes=[pltpu.VMEM((TOPK * nb, n_exp), jnp.float32)],
    )(x2d, wg)


def _dispatch(x2d, e0, e1, l0, l1, g0, g1, offs, cap, n_exp, nb_r):
    t, d = x2d.shape
    rows_out = (n_exp + 1) * cap
    tok_per_w = t // NW
    nch = tok_per_w // DISP_CH
    blk_per_r = RBLK // tok_per_w
    mesh = plsc.VectorSubcoreMesh(core_axis_name="c", subcore_axis_name="s")

    @functools.partial(
        pl.kernel,
        out_type=[
            jax.ShapeDtypeStruct((rows_out, d), jnp.float32),
            jax.ShapeDtypeStruct((t // 16, 16), jnp.int32),
            jax.ShapeDtypeStruct((t // 16, 16), jnp.int32),
            jax.ShapeDtypeStruct((t // 16, 16), jnp.float32),
            jax.ShapeDtypeStruct((t // 16, 16), jnp.float32),
        ],
        mesh=mesh,
        scratch_types=[
            pltpu.VMEM((16,), jnp.float32),
            pltpu.VMEM((16,), jnp.float32),
            pltpu.VMEM((DISP_CH,), jnp.int32),
            pltpu.VMEM((DISP_CH,), jnp.int32),
            pltpu.VMEM((DISP_CH,), jnp.float32),
            pltpu.VMEM((DISP_CH, 1024), jnp.float32),
            pltpu.VMEM((16, 16), jnp.int32),
            pltpu.VMEM((16, 16), jnp.float32),
            pltpu.VMEM((4, 16), jnp.int32),
            pltpu.SemaphoreType.DMA,
            pltpu.SemaphoreType.DMA,
        ],
    )
    def disp_k(x_hbm, e0_hbm, e1_hbm, l0_hbm, l1_hbm, g0_hbm, g1_hbm,
               offs_hbm, out_hbm, pc0_hbm, pc1_hbm, ge0_hbm, ge1_hbm,
               off0_v, off1_v, ev_v, lv_v, gv_v, rows_v, pc_acc, ge_acc,
               pd_buf, s0, s1):
        wid = lax.axis_index("s") * NC + lax.axis_index("c")
        rb = wid // blk_per_r
        pltpu.sync_copy(offs_hbm.at[pl.ds(rb * 16, 16)], off0_v)
        pltpu.sync_copy(offs_hbm.at[pl.ds((nb_r + rb) * 16, 16)], off1_v)
        offs_regs = (off0_v[...], off1_v[...])
        e_hbms = (e0_hbm, e1_hbm)
        l_hbms = (l0_hbm, l1_hbm)
        g_hbms = (g0_hbm, g1_hbm)
        for ch in range(nch):
            base = wid * tok_per_w + ch * DISP_CH
            pltpu.sync_copy(x_hbm.at[pl.ds(base, DISP_CH)], rows_v)
            copies = []
            for j in range(TOPK):
                pltpu.sync_copy(e_hbms[j].at[pl.ds(base, DISP_CH)], ev_v)
                pltpu.sync_copy(l_hbms[j].at[pl.ds(base, DISP_CH)], lv_v)
                pltpu.sync_copy(g_hbms[j].at[pl.ds(base, DISP_CH)], gv_v)
                for grp in range(DISP_CH // 16):
                    ev = ev_v[pl.ds(grp * 16, 16)]
                    lv = lv_v[pl.ds(grp * 16, 16)]
                    gv = gv_v[pl.ds(grp * 16, 16)]
                    offv = offs_regs[j].at[ev].get(mode="promise_in_bounds")
                    loc = lv + offv.astype(jnp.int32)
                    valid = loc < cap
                    pos = ev * cap + jnp.minimum(loc, cap - 1)
                    pdv = jnp.where(valid, pos, n_exp * cap)
                    pcv = jnp.where(valid, pos, 0)
                    gev = jnp.where(valid, gv, 0.0)
                    a0 = j * 8 + ch * (DISP_CH // 16) + grp
                    pc_acc[a0] = pcv
                    ge_acc[a0] = gev
                    kk = j * 2 + grp
                    pd_buf[kk] = pdv
                    copies.append(pltpu.async_copy(
                        rows_v.at[pl.ds(grp * 16, 16)],
                        out_hbm.at[pd_buf.at[kk]], (s0, s1)[j]))
            for c in copies:
                c.wait()
        pltpu.sync_copy(pc_acc.at[pl.ds(0, 8)],
                        pc0_hbm.at[pl.ds(wid * 8, 8)])
        pltpu.sync_copy(pc_acc.at[pl.ds(8, 8)],
                        pc1_hbm.at[pl.ds(wid * 8, 8)])
        pltpu.sync_copy(ge_acc.at[pl.ds(0, 8)],
                        ge0_hbm.at[pl.ds(wid * 8, 8)])
        pltpu.sync_copy(ge_acc.at[pl.ds(8, 8)],
                        ge1_hbm.at[pl.ds(wid * 8, 8)])

    return disp_k(x2d, e0, e1, l0, l1, g0, g1, offs)


def _ffn_body(nh, d_ref, w1_ref, b1_ref, w2_ref, b2_ref, fil_ref, y_ref):
    h = pl.program_id(1)
    w1b = w1_ref[0].astype(jnp.bfloat16)
    hb = jnp.dot(d_ref[...].astype(jnp.bfloat16), w1b,
                 preferred_element_type=jnp.float32)
    hb = jnp.maximum(hb + b1_ref[0], 0.0)
    w2b = w2_ref[0].astype(jnp.bfloat16)
    yb = jnp.dot(hb.astype(jnp.bfloat16), w2b,
                 preferred_element_type=jnp.float32)

    @pl.when(h == 0)
    def _():
        y_ref[...] = yb + b2_ref[0]

    @pl.when(jnp.logical_and(h != 0, h != nh - 1))
    def _():
        y_ref[...] += yb

    @pl.when(h == nh - 1)
    def _():
        acc = y_ref[...] + yb
        rows = lax.broadcasted_iota(jnp.int32, acc.shape, 0)
        y_ref[...] = jnp.where(rows < fil_ref[0, 0, 0], acc, 0.0)


def _ffn(disp, w1, b1, w2, b2, filled, cap):
    n_exp, d, hdim = w1.shape
    nh = hdim // HBLK
    return pl.pallas_call(
        functools.partial(_ffn_body, nh),
        grid=(n_exp, nh),
        in_specs=[
            pl.BlockSpec((cap, d), lambda e, h: (e, 0)),
            pl.BlockSpec((1, d, HBLK), lambda e, h: (e, 0, h)),
            pl.BlockSpec((1, 1, HBLK), lambda e, h: (e, 0, h)),
            pl.BlockSpec((1, HBLK, d), lambda e, h: (e, h, 0)),
            pl.BlockSpec((1, 1, d), lambda e, h: (e, 0, 0)),
            pl.BlockSpec((1, 1, 1), lambda e, h: (e, 0, 0)),
        ],
        out_specs=pl.BlockSpec((cap, d), lambda e, h: (e, 0)),
        out_shape=jax.ShapeDtypeStruct((n_exp * cap, d), jnp.float32),
    )(disp, w1, b1.reshape(n_exp, 1, hdim), w2, b2.reshape(n_exp, 1, d),
      filled)


def _combine_sc(yexp, pc0, pc1, ge0, ge1, t):
    _, d = yexp.shape
    tok_per_w = t // NW
    nch = tok_per_w // 16
    mesh = plsc.VectorSubcoreMesh(core_axis_name="c", subcore_axis_name="s")

    @functools.partial(
        pl.kernel,
        out_type=jax.ShapeDtypeStruct((t, d), jnp.float32),
        mesh=mesh,
        scratch_types=[
            pltpu.VMEM((nch, 16), jnp.int32),
            pltpu.VMEM((nch, 16), jnp.int32),
            pltpu.VMEM((nch, 16), jnp.float32),
            pltpu.VMEM((nch, 16), jnp.float32),
            pltpu.VMEM((2, 16, 1024), jnp.float32),
            pltpu.VMEM((2, 16, 1024), jnp.float32),
            pltpu.VMEM((2, 16, 1024), jnp.float32),
            pltpu.SemaphoreType.DMA,
            pltpu.SemaphoreType.DMA,
            pltpu.SemaphoreType.DMA,
            pltpu.SemaphoreType.DMA,
            pltpu.SemaphoreType.DMA,
        ],
    )
    def comb_k(y_hbm, p0_hbm, p1_hbm, g0_hbm, g1_hbm, out_hbm,
               i0_v, i1_v, g0_v, g1_v, r0_v, r1_v, ro_v, sa0, sa1, sb0, sb1,
               so):
        wid = lax.axis_index("s") * NC + lax.axis_index("c")
        rowb = wid * nch
        pltpu.sync_copy(p0_hbm.at[pl.ds(rowb, nch)], i0_v)
        pltpu.sync_copy(p1_hbm.at[pl.ds(rowb, nch)], i1_v)
        pltpu.sync_copy(g0_hbm.at[pl.ds(rowb, nch)], g0_v)
        pltpu.sync_copy(g1_hbm.at[pl.ds(rowb, nch)], g1_v)
        sems = ((sa0, sa1), (sb0, sb1))

        def start(ch):
            bi = ch % 2
            c0 = pltpu.async_copy(y_hbm.at[i0_v.at[ch]], r0_v.at[bi],
                                  sems[bi][0])
            c1 = pltpu.async_copy(y_hbm.at[i1_v.at[ch]], r1_v.at[bi],
                                  sems[bi][1])
            return (c0, c1)

        pending = start(0)
        out_pending = None
        for ch in range(nch):
            bi = ch % 2
            pending[0].wait()
            pending[1].wait()
            if ch + 1 < nch:
                pending = start(ch + 1)
            ga16 = g0_v[ch]
            gb16 = g1_v[ch]
            if out_pending is not None:
                out_pending.wait()
            for lane in range(16):
                lidx = jnp.full((16,), lane, jnp.int32)
                va = ga16.at[lidx].get(mode="promise_in_bounds")
                vb = gb16.at[lidx].get(mode="promise_in_bounds")

                def col_body(k, c2, lane=lane, va=va, vb=vb, bi=bi):
                    for u in range(4):
                        sl = pl.ds(k * 64 + u * 16, 16)
                        ro_v[bi, lane, sl] = (va * r0_v[bi, lane, sl]
                                              + vb * r1_v[bi, lane, sl])
                    return c2

                lax.fori_loop(0, d // 64, col_body, 0)
            out_pending = pltpu.async_copy(
                ro_v.at[bi], out_hbm.at[pl.ds(wid * tok_per_w + ch * 16, 16)],
                so)
        out_pending.wait()

    return comb_k(yexp, pc0, pc1, ge0, ge1)


def kernel(x, wg, w1, b1, w2, b2):
    orig_shape = x.shape
    d = orig_shape[-1]
    x2d = x.reshape(-1, d)
    t = x2d.shape[0]
    n_exp = w1.shape[0]
    cap = -(-TOPK * t // n_exp)

    e0, e1, l0, l1, g0, g1, offs, filled = _route(x2d, wg, cap)
    nb_r = t // RBLK
    disp, pc0, pc1, ge0, ge1 = _dispatch(
        x2d, e0.reshape(t), e1.reshape(t), l0.reshape(t), l1.reshape(t),
        g0.reshape(t), g1.reshape(t), offs.reshape(-1), cap, n_exp, nb_r)
    yexp = _ffn(disp, w1, b1, w2, b2, filled.reshape(n_exp, 1, 1), cap)
    out = _combine_sc(yexp, pc0, pc1, ge0, ge1, t)
    return out.reshape(orig_shape)
